# baseline (speedup 1.0000x reference)
import jax
import jax.numpy as jnp
from jax.experimental import pallas as pl


def kernel(mesh_points, u, cell_node_map):
    N = mesh_points.shape[0]
    T = cell_node_map.shape[0]
    tri = mesh_points[cell_node_map]
    ones = jnp.ones((T, 3, 1), dtype=mesh_points.dtype)
    A = jnp.concatenate((ones, tri), axis=2)
    B = jnp.broadcast_to(jnp.eye(3, dtype=mesh_points.dtype), (T, 3, 3))
    slopes = jnp.linalg.solve(A, B)
    x = tri[:, :, 0]
    y = tri[:, :, 1]
    area = 0.5 * jnp.abs(
        x[:, 0] * (y[:, 1] - y[:, 2])
        + x[:, 1] * (y[:, 2] - y[:, 0])
        + x[:, 2] * (y[:, 0] - y[:, 1])
    )
    s_i = slopes[:, 1:, 0]
    s_j = slopes[:, 1:, 1]
    s_k = slopes[:, 1:, 2]
    ar = area[:, None]
    Mii = (s_i * s_i * ar).sum(1)
    Mjj = (s_j * s_j * ar).sum(1)
    Mkk = (s_k * s_k * ar).sum(1)
    Mij = (s_i * s_j * ar).sum(1)
    Mjk = (s_j * s_k * ar).sum(1)
    Mki = (s_k * s_i * ar).sum(1)
    i_idx = cell_node_map[:, 0]
    j_idx = cell_node_map[:, 1]
    k_idx = cell_node_map[:, 2]
    row_idx = jnp.concatenate((i_idx, j_idx, k_idx, i_idx, j_idx, k_idx, j_idx, k_idx, i_idx))
    col_idx = jnp.concatenate((i_idx, j_idx, k_idx, j_idx, k_idx, i_idx, i_idx, j_idx, k_idx))
    values = -jnp.concatenate((Mii, Mjj, Mkk, Mij, Mjk, Mki, Mij, Mjk, Mki))
    out = jnp.zeros((N,), dtype=mesh_points.dtype).at[row_idx].add(values * u[col_idx])
    return out


# speedup vs baseline: 410.2928x; 410.2928x over previous
import functools

import jax
import jax.numpy as jnp
from jax import lax
from jax.experimental import pallas as pl
from jax.experimental.pallas import tpu as pltpu
from jax.experimental.pallas import tpu_sc as plsc

N_NODES_PAD = 100352
CHUNK = 128
LANES = 16


def _fem_body(xs_hbm, ys_hbm, us_hbm, cmap_hbm, zeros_hbm, out_hbm, *,
              n_cells, cpt, n_chunks, idx_v, gx, gy, gu,
              val_i, val_j, val_k, out_buf, acc_sh, sem):
    nc = 2
    ns = 16
    cid = lax.axis_index("c")
    sid = lax.axis_index("s")
    wid = sid * nc + cid

    pltpu.sync_copy(cmap_hbm.at[0, wid], idx_v.at[0])
    pltpu.sync_copy(cmap_hbm.at[1, wid], idx_v.at[1])
    pltpu.sync_copy(cmap_hbm.at[2, wid], idx_v.at[2])

    slice_n = N_NODES_PAD // ns
    pltpu.sync_copy(zeros_hbm, acc_sh.at[pl.ds(sid * slice_n, slice_n)])
    plsc.subcore_barrier()

    lane_iota = lax.iota(jnp.int32, LANES)
    nanc = jnp.full((LANES,), jnp.nan, jnp.float32)
    zeroc = jnp.zeros((LANES,), jnp.float32)
    halfc = jnp.full((LANES,), 0.5, jnp.float32)
    tile_base = wid * cpt

    def chunk_body(ch, carry):
        for v, bufs in ((0, (gx.at[0], gy.at[0], gu.at[0])),
                        (1, (gx.at[1], gy.at[1], gu.at[1])),
                        (2, (gx.at[2], gy.at[2], gu.at[2]))):
            ids = idx_v.at[v, ch]
            pltpu.async_copy(xs_hbm.at[ids], bufs[0], sem).wait()
            pltpu.async_copy(ys_hbm.at[ids], bufs[1], sem).wait()
            pltpu.async_copy(us_hbm.at[ids], bufs[2], sem).wait()

        for s in range(CHUNK // LANES):
            sl = pl.ds(s * LANES, LANES)
            xi = gx[0, sl]
            yi = gy[0, sl]
            ui = gu[0, sl]
            xj = gx[1, sl]
            yj = gy[1, sl]
            uj = gu[1, sl]
            xk = gx[2, sl]
            yk = gy[2, sl]
            uk = gu[2, sl]

            bi = yj - yk
            bj = yk - yi
            bk = yi - yj
            ci = xk - xj
            cj = xi - xk
            ck = xj - xi
            det = ck * bj - cj * bk
            inv = halfc / jnp.abs(det)
            inv = jnp.where(det == zeroc, nanc, inv)
            mii = (bi * bi + ci * ci) * inv
            mjj = (bj * bj + cj * cj) * inv
            mkk = (bk * bk + ck * ck) * inv
            mij = (bi * bj + ci * cj) * inv
            mjk = (bj * bk + cj * ck) * inv
            mki = (bk * bi + ck * ci) * inv
            vi = -(mii * ui + mij * uj + mki * uk)
            vj = -(mij * ui + mjj * uj + mjk * uk)
            vk = -(mki * ui + mjk * uj + mkk * uk)
            r = lane_iota + s * LANES
            valid = (tile_base + ch * CHUNK + r) < n_cells
            vi = jnp.where(valid, vi, zeroc)
            vj = jnp.where(valid, vj, zeroc)
            vk = jnp.where(valid, vk, zeroc)
            val_i[sl] = vi
            val_j[sl] = vj
            val_k[sl] = vk

        pltpu.sync_copy(val_i, acc_sh.at[idx_v.at[0, ch]], add=True)
        pltpu.sync_copy(val_j, acc_sh.at[idx_v.at[1, ch]], add=True)
        pltpu.sync_copy(val_k, acc_sh.at[idx_v.at[2, ch]], add=True)
        return carry

    lax.fori_loop(0, n_chunks, chunk_body, 0, unroll=False)

    plsc.subcore_barrier()
    pltpu.sync_copy(acc_sh.at[pl.ds(sid * slice_n, slice_n)], out_buf)
    pltpu.sync_copy(out_buf, out_hbm.at[cid, pl.ds(sid * slice_n, slice_n)])


def _build_sc_call(n_cells, cpt, n_chunks):
    mesh = plsc.VectorSubcoreMesh(core_axis_name="c", subcore_axis_name="s")
    slice_n = N_NODES_PAD // 16
    body = functools.partial(_fem_body, n_cells=n_cells, cpt=cpt,
                             n_chunks=n_chunks)
    return pl.kernel(
        body,
        out_type=jax.ShapeDtypeStruct((2, N_NODES_PAD), jnp.float32),
        mesh=mesh,
        scratch_types=dict(
            idx_v=pltpu.VMEM((3, n_chunks, CHUNK), jnp.int32),
            gx=pltpu.VMEM((3, CHUNK), jnp.float32),
            gy=pltpu.VMEM((3, CHUNK), jnp.float32),
            gu=pltpu.VMEM((3, CHUNK), jnp.float32),
            val_i=pltpu.VMEM((CHUNK,), jnp.float32),
            val_j=pltpu.VMEM((CHUNK,), jnp.float32),
            val_k=pltpu.VMEM((CHUNK,), jnp.float32),
            out_buf=pltpu.VMEM((slice_n,), jnp.float32),
            acc_sh=pltpu.VMEM_SHARED((N_NODES_PAD,), jnp.float32),
            sem=pltpu.SemaphoreType.DMA,
        ),
    )


def kernel(mesh_points, u, cell_node_map):
    n = mesh_points.shape[0]
    t = cell_node_map.shape[0]
    n_tiles = 32
    cpt = -(-t // (n_tiles * CHUNK)) * CHUNK
    n_chunks = cpt // CHUNK
    slice_n = N_NODES_PAD // 16

    mp = mesh_points.astype(jnp.float32)
    xs = mp[:, 0]
    ys = mp[:, 1]
    us = u.astype(jnp.float32)

    idx = cell_node_map.astype(jnp.int32).T
    pad = n_tiles * cpt - t
    idx = jnp.pad(idx, ((0, 0), (0, pad)))
    idx = idx.reshape(3, n_tiles, n_chunks, CHUNK)
    zeros = jnp.zeros((slice_n,), jnp.float32)

    call = _build_sc_call(t, cpt, n_chunks)
    partials = call(xs, ys, us, idx, zeros)
    out = partials[0] + partials[1]
    return out[:n]


# speedup vs baseline: 765.1094x; 1.8648x over previous
import functools

import jax
import jax.numpy as jnp
from jax import lax
from jax.experimental import pallas as pl
from jax.experimental.pallas import tpu as pltpu
from jax.experimental.pallas import tpu_sc as plsc

N_NODES_PAD = 100352
CHUNK = 128
LANES = 16


def _fem_body(xs_hbm, ys_hbm, us_hbm, cmap_hbm, zeros_hbm, out_hbm, *,
              n_cells, cpt, n_chunks, idx_v, gbuf,
              val_i, val_j, val_k, out_buf, acc_sh, sem):
    nc = 2
    ns = 16
    cid = lax.axis_index("c")
    sid = lax.axis_index("s")
    wid = sid * nc + cid

    pltpu.sync_copy(cmap_hbm.at[0, wid], idx_v.at[0])
    pltpu.sync_copy(cmap_hbm.at[1, wid], idx_v.at[1])
    pltpu.sync_copy(cmap_hbm.at[2, wid], idx_v.at[2])

    slice_n = N_NODES_PAD // ns
    pltpu.sync_copy(zeros_hbm, acc_sh.at[pl.ds(sid * slice_n, slice_n)])
    plsc.subcore_barrier()

    lane_iota = lax.iota(jnp.int32, LANES)
    nanc = jnp.full((LANES,), jnp.nan, jnp.float32)
    zeroc = jnp.zeros((LANES,), jnp.float32)
    halfc = jnp.full((LANES,), 0.5, jnp.float32)
    tile_base = wid * cpt
    tabs = (xs_hbm, ys_hbm, us_hbm)

    def issue(ch, parity):
        for v in range(3):
            ids = idx_v.at[v, ch]
            for tb in range(3):
                pltpu.async_copy(tabs[tb].at[ids], gbuf.at[parity, v, tb], sem)

    def drain():
        for _ in range(9):
            pltpu.make_async_copy(
                xs_hbm.at[pl.ds(0, CHUNK)], gbuf.at[0, 0, 0], sem).wait()

    issue(0, 0)

    def chunk_body(ch, carry):
        p = lax.rem(ch, 2)
        drain()
        issue(ch + 1, 1 - p)

        for s in range(CHUNK // LANES):
            sl = pl.ds(s * LANES, LANES)
            xi = gbuf[p, 0, 0, sl]
            yi = gbuf[p, 0, 1, sl]
            ui = gbuf[p, 0, 2, sl]
            xj = gbuf[p, 1, 0, sl]
            yj = gbuf[p, 1, 1, sl]
            uj = gbuf[p, 1, 2, sl]
            xk = gbuf[p, 2, 0, sl]
            yk = gbuf[p, 2, 1, sl]
            uk = gbuf[p, 2, 2, sl]

            bi = yj - yk
            bj = yk - yi
            bk = yi - yj
            ci = xk - xj
            cj = xi - xk
            ck = xj - xi
            det = ck * bj - cj * bk
            inv = halfc / jnp.abs(det)
            inv = jnp.where(det == zeroc, nanc, inv)
            g1 = (bi * ui + bj * uj + bk * uk) * inv
            g2 = (ci * ui + cj * uj + ck * uk) * inv
            vi = -(bi * g1 + ci * g2)
            vj = -(bj * g1 + cj * g2)
            vk = -(bk * g1 + ck * g2)
            r = lane_iota + s * LANES
            valid = (tile_base + ch * CHUNK + r) < n_cells
            vi = jnp.where(valid, vi, zeroc)
            vj = jnp.where(valid, vj, zeroc)
            vk = jnp.where(valid, vk, zeroc)
            val_i[sl] = vi
            val_j[sl] = vj
            val_k[sl] = vk

        pltpu.sync_copy(val_i, acc_sh.at[idx_v.at[0, ch]], add=True)
        pltpu.sync_copy(val_j, acc_sh.at[idx_v.at[1, ch]], add=True)
        pltpu.sync_copy(val_k, acc_sh.at[idx_v.at[2, ch]], add=True)
        return carry

    lax.fori_loop(0, n_chunks, chunk_body, 0, unroll=False)
    drain()

    plsc.subcore_barrier()
    pltpu.sync_copy(acc_sh.at[pl.ds(sid * slice_n, slice_n)], out_buf)
    pltpu.sync_copy(out_buf, out_hbm.at[cid, pl.ds(sid * slice_n, slice_n)])


def _build_sc_call(n_cells, cpt, n_chunks):
    mesh = plsc.VectorSubcoreMesh(core_axis_name="c", subcore_axis_name="s")
    slice_n = N_NODES_PAD // 16
    body = functools.partial(_fem_body, n_cells=n_cells, cpt=cpt,
                             n_chunks=n_chunks)
    return pl.kernel(
        body,
        out_type=jax.ShapeDtypeStruct((2, N_NODES_PAD), jnp.float32),
        mesh=mesh,
        scratch_types=dict(
            idx_v=pltpu.VMEM((3, n_chunks + 1, CHUNK), jnp.int32),
            gbuf=pltpu.VMEM((2, 3, 3, CHUNK), jnp.float32),
            val_i=pltpu.VMEM((CHUNK,), jnp.float32),
            val_j=pltpu.VMEM((CHUNK,), jnp.float32),
            val_k=pltpu.VMEM((CHUNK,), jnp.float32),
            out_buf=pltpu.VMEM((slice_n,), jnp.float32),
            acc_sh=pltpu.VMEM_SHARED((N_NODES_PAD,), jnp.float32),
            sem=pltpu.SemaphoreType.DMA,
        ),
    )


def kernel(mesh_points, u, cell_node_map):
    n = mesh_points.shape[0]
    t = cell_node_map.shape[0]
    n_tiles = 32
    cpt = -(-t // (n_tiles * CHUNK)) * CHUNK
    n_chunks = cpt // CHUNK
    slice_n = N_NODES_PAD // 16

    mp = mesh_points.astype(jnp.float32)
    xs = mp[:, 0]
    ys = mp[:, 1]
    us = u.astype(jnp.float32)

    idx = cell_node_map.astype(jnp.int32).T
    pad = n_tiles * cpt - t
    idx = jnp.pad(idx, ((0, 0), (0, pad)))
    idx = idx.reshape(3, n_tiles, n_chunks, CHUNK)
    idx = jnp.concatenate(
        [idx, jnp.zeros((3, n_tiles, 1, CHUNK), jnp.int32)], axis=2)
    zeros = jnp.zeros((slice_n,), jnp.float32)

    call = _build_sc_call(t, cpt, n_chunks)
    partials = call(xs, ys, us, idx, zeros)
    out = partials[0] + partials[1]
    return out[:n]


# speedup vs baseline: 1841.8827x; 2.4073x over previous
import functools

import jax
import jax.numpy as jnp
from jax import lax
from jax.experimental import pallas as pl
from jax.experimental.pallas import tpu as pltpu
from jax.experimental.pallas import tpu_sc as plsc

N_NODES_PAD = 100352
CHUNK = 128
LANES = 16


def _fem_body(xs_hbm, ys_hbm, us_hbm, cmap_hbm, zeros_hbm, out_hbm, *,
              n_cells, cpt, n_chunks, idx_v, gbuf,
              val_i, val_j, val_k, out_buf, acc_sh, xs_sh, ys_sh, us_sh, sem):
    nc = 2
    ns = 16
    cid = lax.axis_index("c")
    sid = lax.axis_index("s")
    wid = sid * nc + cid

    pltpu.sync_copy(cmap_hbm.at[0, wid], idx_v.at[0])
    pltpu.sync_copy(cmap_hbm.at[1, wid], idx_v.at[1])
    pltpu.sync_copy(cmap_hbm.at[2, wid], idx_v.at[2])

    slice_n = N_NODES_PAD // ns
    tsl = pl.ds(sid * slice_n, slice_n)
    pltpu.sync_copy(xs_hbm.at[tsl], xs_sh.at[tsl])
    pltpu.sync_copy(ys_hbm.at[tsl], ys_sh.at[tsl])
    pltpu.sync_copy(us_hbm.at[tsl], us_sh.at[tsl])
    pltpu.sync_copy(zeros_hbm, acc_sh.at[tsl])
    plsc.subcore_barrier()

    lane_iota = lax.iota(jnp.int32, LANES)
    nanc = jnp.full((LANES,), jnp.nan, jnp.float32)
    zeroc = jnp.zeros((LANES,), jnp.float32)
    halfc = jnp.full((LANES,), 0.5, jnp.float32)
    tile_base = wid * cpt
    tabs = (xs_sh, ys_sh, us_sh)

    def issue(ch, parity):
        for v in range(3):
            ids = idx_v.at[v, ch]
            for tb in range(3):
                pltpu.async_copy(tabs[tb].at[ids], gbuf.at[parity, v, tb], sem)

    def drain():
        for _ in range(9):
            pltpu.make_async_copy(
                xs_hbm.at[pl.ds(0, CHUNK)], gbuf.at[0, 0, 0], sem).wait()

    issue(0, 0)

    def chunk_body(ch, carry):
        p = lax.rem(ch, 2)
        drain()
        issue(ch + 1, 1 - p)

        for s in range(CHUNK // LANES):
            sl = pl.ds(s * LANES, LANES)
            xi = gbuf[p, 0, 0, sl]
            yi = gbuf[p, 0, 1, sl]
            ui = gbuf[p, 0, 2, sl]
            xj = gbuf[p, 1, 0, sl]
            yj = gbuf[p, 1, 1, sl]
            uj = gbuf[p, 1, 2, sl]
            xk = gbuf[p, 2, 0, sl]
            yk = gbuf[p, 2, 1, sl]
            uk = gbuf[p, 2, 2, sl]

            bi = yj - yk
            bj = yk - yi
            bk = yi - yj
            ci = xk - xj
            cj = xi - xk
            ck = xj - xi
            det = ck * bj - cj * bk
            inv = halfc / jnp.abs(det)
            inv = jnp.where(det == zeroc, nanc, inv)
            g1 = (bi * ui + bj * uj + bk * uk) * inv
            g2 = (ci * ui + cj * uj + ck * uk) * inv
            vi = -(bi * g1 + ci * g2)
            vj = -(bj * g1 + cj * g2)
            vk = -(bk * g1 + ck * g2)
            r = lane_iota + s * LANES
            valid = (tile_base + ch * CHUNK + r) < n_cells
            vi = jnp.where(valid, vi, zeroc)
            vj = jnp.where(valid, vj, zeroc)
            vk = jnp.where(valid, vk, zeroc)
            val_i[sl] = vi
            val_j[sl] = vj
            val_k[sl] = vk

        pltpu.sync_copy(val_i, acc_sh.at[idx_v.at[0, ch]], add=True)
        pltpu.sync_copy(val_j, acc_sh.at[idx_v.at[1, ch]], add=True)
        pltpu.sync_copy(val_k, acc_sh.at[idx_v.at[2, ch]], add=True)
        return carry

    lax.fori_loop(0, n_chunks, chunk_body, 0, unroll=False)
    drain()

    plsc.subcore_barrier()
    pltpu.sync_copy(acc_sh.at[pl.ds(sid * slice_n, slice_n)], out_buf)
    pltpu.sync_copy(out_buf, out_hbm.at[cid, pl.ds(sid * slice_n, slice_n)])


def _build_sc_call(n_cells, cpt, n_chunks):
    mesh = plsc.VectorSubcoreMesh(core_axis_name="c", subcore_axis_name="s")
    slice_n = N_NODES_PAD // 16
    body = functools.partial(_fem_body, n_cells=n_cells, cpt=cpt,
                             n_chunks=n_chunks)
    return pl.kernel(
        body,
        out_type=jax.ShapeDtypeStruct((2, N_NODES_PAD), jnp.float32),
        mesh=mesh,
        scratch_types=dict(
            idx_v=pltpu.VMEM((3, n_chunks + 1, CHUNK), jnp.int32),
            gbuf=pltpu.VMEM((2, 3, 3, CHUNK), jnp.float32),
            val_i=pltpu.VMEM((CHUNK,), jnp.float32),
            val_j=pltpu.VMEM((CHUNK,), jnp.float32),
            val_k=pltpu.VMEM((CHUNK,), jnp.float32),
            out_buf=pltpu.VMEM((slice_n,), jnp.float32),
            acc_sh=pltpu.VMEM_SHARED((N_NODES_PAD,), jnp.float32),
            xs_sh=pltpu.VMEM_SHARED((N_NODES_PAD,), jnp.float32),
            ys_sh=pltpu.VMEM_SHARED((N_NODES_PAD,), jnp.float32),
            us_sh=pltpu.VMEM_SHARED((N_NODES_PAD,), jnp.float32),
            sem=pltpu.SemaphoreType.DMA,
        ),
    )


def kernel(mesh_points, u, cell_node_map):
    n = mesh_points.shape[0]
    t = cell_node_map.shape[0]
    n_tiles = 32
    cpt = -(-t // (n_tiles * CHUNK)) * CHUNK
    n_chunks = cpt // CHUNK
    slice_n = N_NODES_PAD // 16

    mp = mesh_points.astype(jnp.float32)
    npad = N_NODES_PAD - n
    xs = jnp.pad(mp[:, 0], (0, npad))
    ys = jnp.pad(mp[:, 1], (0, npad))
    us = jnp.pad(u.astype(jnp.float32), (0, npad))

    idx = cell_node_map.astype(jnp.int32).T
    pad = n_tiles * cpt - t
    idx = jnp.pad(idx, ((0, 0), (0, pad)))
    idx = idx.reshape(3, n_tiles, n_chunks, CHUNK)
    idx = jnp.concatenate(
        [idx, jnp.zeros((3, n_tiles, 1, CHUNK), jnp.int32)], axis=2)
    zeros = jnp.zeros((slice_n,), jnp.float32)

    call = _build_sc_call(t, cpt, n_chunks)
    partials = call(xs, ys, us, idx, zeros)
    out = partials[0] + partials[1]
    return out[:n]


# speedup vs baseline: 1847.8507x; 1.0032x over previous
import functools

import jax
import jax.numpy as jnp
from jax import lax
from jax.experimental import pallas as pl
from jax.experimental.pallas import tpu as pltpu
from jax.experimental.pallas import tpu_sc as plsc

N_NODES_PAD = 100352
CHUNK = 128
LANES = 16


def _fem_body(xs_hbm, ys_hbm, us_hbm, cmap_hbm, zeros_hbm, dummy_hbm, out_hbm,
              *, n_cells, cpt, n_chunks, idx_v, gbuf,
              vbuf, out_buf, acc_sh, xs_sh, ys_sh, us_sh, sem, ssem):
    nc = 2
    ns = 16
    cid = lax.axis_index("c")
    sid = lax.axis_index("s")
    wid = sid * nc + cid

    pltpu.sync_copy(cmap_hbm.at[0, wid], idx_v.at[0])
    pltpu.sync_copy(cmap_hbm.at[1, wid], idx_v.at[1])
    pltpu.sync_copy(cmap_hbm.at[2, wid], idx_v.at[2])

    slice_n = N_NODES_PAD // ns
    tsl = pl.ds(sid * slice_n, slice_n)
    pltpu.sync_copy(xs_hbm.at[tsl], xs_sh.at[tsl])
    pltpu.sync_copy(ys_hbm.at[tsl], ys_sh.at[tsl])
    pltpu.sync_copy(us_hbm.at[tsl], us_sh.at[tsl])
    pltpu.sync_copy(zeros_hbm, acc_sh.at[tsl])
    plsc.subcore_barrier()

    lane_iota = lax.iota(jnp.int32, LANES)
    nanc = jnp.full((LANES,), jnp.nan, jnp.float32)
    zeroc = jnp.zeros((LANES,), jnp.float32)
    halfc = jnp.full((LANES,), 0.5, jnp.float32)
    tile_base = wid * cpt
    tabs = (xs_sh, ys_sh, us_sh)

    def issue(ch, parity):
        for v in range(3):
            ids = idx_v.at[v, ch]
            for tb in range(3):
                pltpu.async_copy(tabs[tb].at[ids], gbuf.at[parity, v, tb], sem)

    def drain_gathers(parity):
        pltpu.make_async_copy(dummy_hbm, gbuf.at[parity], sem).wait()

    def drain_scatters(parity):
        pltpu.make_async_copy(dummy_hbm.at[0], vbuf.at[parity], ssem).wait()

    issue(0, 0)

    def chunk_body(ch, carry):
        p = lax.rem(ch, 2)
        drain_gathers(p)
        issue(ch + 1, 1 - p)

        @pl.when(ch >= 2)
        def _():
            drain_scatters(p)

        for s in range(CHUNK // LANES):
            sl = pl.ds(s * LANES, LANES)
            xi = gbuf[p, 0, 0, sl]
            yi = gbuf[p, 0, 1, sl]
            ui = gbuf[p, 0, 2, sl]
            xj = gbuf[p, 1, 0, sl]
            yj = gbuf[p, 1, 1, sl]
            uj = gbuf[p, 1, 2, sl]
            xk = gbuf[p, 2, 0, sl]
            yk = gbuf[p, 2, 1, sl]
            uk = gbuf[p, 2, 2, sl]

            bi = yj - yk
            bj = yk - yi
            bk = yi - yj
            ci = xk - xj
            cj = xi - xk
            ck = xj - xi
            det = ck * bj - cj * bk
            inv = halfc / jnp.abs(det)
            inv = jnp.where(det == zeroc, nanc, inv)
            g1 = (bi * ui + bj * uj + bk * uk) * inv
            g2 = (ci * ui + cj * uj + ck * uk) * inv
            vi = -(bi * g1 + ci * g2)
            vj = -(bj * g1 + cj * g2)
            vk = -(bk * g1 + ck * g2)
            r = lane_iota + s * LANES
            valid = (tile_base + ch * CHUNK + r) < n_cells
            vi = jnp.where(valid, vi, zeroc)
            vj = jnp.where(valid, vj, zeroc)
            vk = jnp.where(valid, vk, zeroc)
            vbuf[p, 0, sl] = vi
            vbuf[p, 1, sl] = vj
            vbuf[p, 2, sl] = vk

        pltpu.async_copy(vbuf.at[p, 0], acc_sh.at[idx_v.at[0, ch]], ssem, add=True)
        pltpu.async_copy(vbuf.at[p, 1], acc_sh.at[idx_v.at[1, ch]], ssem, add=True)
        pltpu.async_copy(vbuf.at[p, 2], acc_sh.at[idx_v.at[2, ch]], ssem, add=True)
        return carry

    lax.fori_loop(0, n_chunks, chunk_body, 0, unroll=False)
    drain_gathers(n_chunks % 2)
    drain_scatters((n_chunks - 2) % 2)
    drain_scatters((n_chunks - 1) % 2)

    plsc.subcore_barrier()
    pltpu.sync_copy(acc_sh.at[pl.ds(sid * slice_n, slice_n)], out_buf)
    pltpu.sync_copy(out_buf, out_hbm.at[cid, pl.ds(sid * slice_n, slice_n)])


def _build_sc_call(n_cells, cpt, n_chunks):
    mesh = plsc.VectorSubcoreMesh(core_axis_name="c", subcore_axis_name="s")
    slice_n = N_NODES_PAD // 16
    body = functools.partial(_fem_body, n_cells=n_cells, cpt=cpt,
                             n_chunks=n_chunks)
    return pl.kernel(
        body,
        out_type=jax.ShapeDtypeStruct((2, N_NODES_PAD), jnp.float32),
        mesh=mesh,
        scratch_types=dict(
            idx_v=pltpu.VMEM((3, n_chunks + 1, CHUNK), jnp.int32),
            gbuf=pltpu.VMEM((2, 3, 3, CHUNK), jnp.float32),
            vbuf=pltpu.VMEM((2, 3, CHUNK), jnp.float32),
            out_buf=pltpu.VMEM((slice_n,), jnp.float32),
            acc_sh=pltpu.VMEM_SHARED((N_NODES_PAD,), jnp.float32),
            xs_sh=pltpu.VMEM_SHARED((N_NODES_PAD,), jnp.float32),
            ys_sh=pltpu.VMEM_SHARED((N_NODES_PAD,), jnp.float32),
            us_sh=pltpu.VMEM_SHARED((N_NODES_PAD,), jnp.float32),
            sem=pltpu.SemaphoreType.DMA,
            ssem=pltpu.SemaphoreType.DMA,
        ),
    )


def kernel(mesh_points, u, cell_node_map):
    n = mesh_points.shape[0]
    t = cell_node_map.shape[0]
    n_tiles = 32
    cpt = -(-t // (n_tiles * CHUNK)) * CHUNK
    n_chunks = cpt // CHUNK
    slice_n = N_NODES_PAD // 16

    mp = mesh_points.astype(jnp.float32)
    npad = N_NODES_PAD - n
    xs = jnp.pad(mp[:, 0], (0, npad))
    ys = jnp.pad(mp[:, 1], (0, npad))
    us = jnp.pad(u.astype(jnp.float32), (0, npad))

    idx = cell_node_map.astype(jnp.int32).T
    pad = n_tiles * cpt - t
    idx = jnp.pad(idx, ((0, 0), (0, pad)))
    idx = idx.reshape(3, n_tiles, n_chunks, CHUNK)
    idx = jnp.concatenate(
        [idx, jnp.zeros((3, n_tiles, 1, CHUNK), jnp.int32)], axis=2)
    zeros = jnp.zeros((slice_n,), jnp.float32)
    dummy = jnp.zeros((3, 3, CHUNK), jnp.float32)

    call = _build_sc_call(t, cpt, n_chunks)
    partials = call(xs, ys, us, idx, zeros, dummy)
    out = partials[0] + partials[1]
    return out[:n]
